# baseline (speedup 1.0000x reference)
import jax, jax.numpy as jnp
from jax.experimental import pallas as pl

N = 10000
HIDLAYERS = 3
ALPHA_Q = 0.7
ALPHA_P = 0.3


def _gcn_conv(x, edge_index, W, b):
    n = x.shape[0]
    loop = jnp.arange(n, dtype=edge_index.dtype)
    src = jnp.concatenate([edge_index[0], loop])
    dst = jnp.concatenate([edge_index[1], loop])
    deg = jax.ops.segment_sum(jnp.ones(src.shape[0], dtype=x.dtype), dst, num_segments=n)
    dinv = jnp.where(deg > 0, deg ** -0.5, 0.0)
    norm = dinv[src] * dinv[dst]
    h = x @ W
    msg = jnp.take(h, src, axis=0) * norm[:, None]
    out = jax.ops.segment_sum(msg, dst, num_segments=n)
    return out + b


def kernel(x, edge_index_q, edge_index_p, Wq0, bq0, Wp0, bp0, Wq1, bq1, Wp1, bp1, Wq2, bq2, Wp2, bp2):
    mu = x.mean(axis=0, keepdims=True)
    sd = x.std(axis=0, keepdims=True, ddof=1)
    x = (x - mu) / sd
    tq = ALPHA_Q * jax.nn.relu(_gcn_conv(x, edge_index_q, Wq0, bq0))
    tp = ALPHA_P * jax.nn.relu(_gcn_conv(x, edge_index_p, Wp0, bp0))
    x = tq + tp
    for _ in range(HIDLAYERS - 1):
        tq = ALPHA_Q * jax.nn.relu(_gcn_conv(x, edge_index_q, Wq1, bq1))
        tp = ALPHA_P * jax.nn.relu(_gcn_conv(x, edge_index_p, Wp1, bp1))
        x = tq + tp
    x = ALPHA_Q * _gcn_conv(x, edge_index_q, Wq2, bq2) + ALPHA_P * _gcn_conv(x, edge_index_p, Wp2, bp2)
    return jax.nn.log_softmax(x, axis=1)


# speedup vs baseline: 7.6732x; 7.6732x over previous
import functools

import jax
import jax.numpy as jnp
from jax import lax
from jax.experimental import pallas as pl
from jax.experimental.pallas import tpu as pltpu
from jax.experimental.pallas import tpu_sc as plsc

N = 10000
NP = 10240
NPT = NP // 16
E = 320000
EPT = E // 16
NB = 157
EPTP = NB * 128
HIDLAYERS = 3
ALPHA_Q = 0.7
ALPHA_P = 0.3

_MESH = plsc.VectorSubcoreMesh(core_axis_name="c", subcore_axis_name="s")


@functools.partial(
    pl.kernel,
    out_type=jax.ShapeDtypeStruct((2, NP, 16), jnp.float32),
    mesh=_MESH,
    compiler_params=pltpu.CompilerParams(use_tc_tiling_on_sc=False),
    scratch_types=[
        pltpu.VMEM((NB, 128), jnp.int32),
        pltpu.VMEM((5, 128), jnp.int32),
        pltpu.VMEM((128, 16), jnp.float32),
        pltpu.VMEM((NPT, 16), jnp.float32),
        pltpu.VMEM_SHARED((NP, 16), jnp.float32),
    ],
)
def _deg_kernel(dst_hbm, zi_hbm, deg_out, idx_v, zi_v, ones_v, stage_v, deg_sh):
    c = lax.axis_index("c")
    s = lax.axis_index("s")

    pltpu.sync_copy(dst_hbm.at[c, s], idx_v)
    pltpu.sync_copy(zi_hbm.at[s], zi_v)

    ones16 = jnp.ones((16,), jnp.float32)
    zeros16 = jnp.zeros((16,), jnp.float32)

    @pl.loop(0, 128)
    def _(i):
        ones_v[i, :] = ones16

    @pl.loop(0, NPT)
    def _(i):
        stage_v[i, :] = zeros16

    for k in range(5):
        pltpu.sync_copy(stage_v.at[pl.ds(0, 128)], deg_sh.at[zi_v.at[k]])
    plsc.subcore_barrier()

    @pl.loop(0, NB)
    def _(j):
        pltpu.sync_copy(ones_v, deg_sh.at[idx_v.at[j]], add=True)

    plsc.subcore_barrier()
    for k in range(5):
        pltpu.sync_copy(deg_sh.at[zi_v.at[k]], stage_v.at[pl.ds(k * 128, 128)])
    pltpu.sync_copy(stage_v, deg_out.at[c, pl.ds(s * NPT, NPT)])


def _make_conv(W, ROUNDS):
    CH = 2 * ROUNDS

    @functools.partial(
        pl.kernel,
        out_type=jax.ShapeDtypeStruct((CH, NP, W), jnp.float32),
        mesh=_MESH,
        compiler_params=pltpu.CompilerParams(use_tc_tiling_on_sc=False),
        scratch_types=[
            pltpu.VMEM((NB, 128), jnp.int32),
            pltpu.VMEM((NB, 128), jnp.int32),
            pltpu.VMEM((5, 128), jnp.int32),
            pltpu.VMEM((128, W), jnp.float32),
            pltpu.VMEM_SHARED((NP, W), jnp.float32),
        ],
    )
    def conv(h_hbm, src_hbm, dst_hbm, zi_hbm, out_hbm, src_v, dst_v, zi_v, buf_v, agg_sh):
        c = lax.axis_index("c")
        s = lax.axis_index("s")
        pltpu.sync_copy(dst_hbm.at[s], dst_v)
        pltpu.sync_copy(zi_hbm.at[s], zi_v)
        zeros16 = jnp.zeros((16,), jnp.float32)

        for r in range(ROUNDS):
            ch = c + 2 * r
            pltpu.sync_copy(src_hbm.at[ch, s], src_v)

            @pl.loop(0, 128)
            def _(i):
                for t in range(W // 16):
                    buf_v[i, pl.ds(t * 16, 16)] = zeros16

            for k in range(5):
                pltpu.sync_copy(buf_v, agg_sh.at[zi_v.at[k]])
            plsc.subcore_barrier()

            @pl.loop(0, NB)
            def _(j):
                pltpu.sync_copy(h_hbm.at[src_v.at[j]], buf_v)
                pltpu.sync_copy(buf_v, agg_sh.at[dst_v.at[j]], add=True)

            plsc.subcore_barrier()
            for k in range(5):
                pltpu.sync_copy(agg_sh.at[zi_v.at[k]], buf_v)
                pltpu.sync_copy(buf_v, out_hbm.at[ch, pl.ds(s * NPT + k * 128, 128)])
            if r + 1 < ROUNDS:
                plsc.subcore_barrier()

    return conv


_conv64 = _make_conv(64, 2)
_conv32 = _make_conv(32, 1)


def _pad_edge(e, fill):
    d = e.reshape(16, EPT)
    d = jnp.pad(d, ((0, 0), (0, EPTP - EPT)), constant_values=fill)
    return d.reshape(16, NB, 128)


def _sc_conv(h, dinv, src_off, dst_t, zi, conv_fn, W):
    CH = h.shape[1] // W
    hp = h * dinv[:, None]
    hp_pad = jnp.pad(hp, ((0, NP - N), (0, 0)))
    hchunk = jnp.concatenate([hp_pad[:, k * W:(k + 1) * W] for k in range(CH)], axis=0)
    agg = conv_fn(hchunk, src_off, dst_t, zi)
    agg_full = jnp.concatenate([agg[k, :N] for k in range(CH)], axis=1)
    return (agg_full + hp) * dinv[:, None]


def kernel(x, edge_index_q, edge_index_p, Wq0, bq0, Wp0, bp0, Wq1, bq1, Wp1, bp1, Wq2, bq2, Wp2, bp2):
    dst2 = jnp.stack([_pad_edge(edge_index_q[1], N), _pad_edge(edge_index_p[1], N)])
    zi = jnp.arange(NP, dtype=jnp.int32).reshape(16, 5, 128)
    deg = _deg_kernel(dst2, zi)
    dinv_q = (deg[0, :N, 0] + 1.0) ** -0.5
    dinv_p = (deg[1, :N, 0] + 1.0) ** -0.5

    sq = _pad_edge(edge_index_q[0], 0)
    sp = _pad_edge(edge_index_p[0], 0)
    src_q4 = jnp.stack([sq + k * NP for k in range(4)])
    src_p4 = jnp.stack([sp + k * NP for k in range(4)])
    src_q2 = src_q4[:2]
    src_p2 = src_p4[:2]
    dst_q = _pad_edge(edge_index_q[1], N)
    dst_p = _pad_edge(edge_index_p[1], N)

    mu = x.mean(axis=0, keepdims=True)
    sd = x.std(axis=0, keepdims=True, ddof=1)
    x = (x - mu) / sd
    cq = _sc_conv(x @ Wq0, dinv_q, src_q4, dst_q, zi, _conv64, 64) + bq0
    cp = _sc_conv(x @ Wp0, dinv_p, src_p4, dst_p, zi, _conv64, 64) + bp0
    x = ALPHA_Q * jax.nn.relu(cq) + ALPHA_P * jax.nn.relu(cp)
    for _ in range(HIDLAYERS - 1):
        cq = _sc_conv(x @ Wq1, dinv_q, src_q4, dst_q, zi, _conv64, 64) + bq1
        cp = _sc_conv(x @ Wp1, dinv_p, src_p4, dst_p, zi, _conv64, 64) + bp1
        x = ALPHA_Q * jax.nn.relu(cq) + ALPHA_P * jax.nn.relu(cp)
    cq = _sc_conv(x @ Wq2, dinv_q, src_q2, dst_q, zi, _conv32, 32) + bq2
    cp = _sc_conv(x @ Wp2, dinv_p, src_p2, dst_p, zi, _conv32, 32) + bp2
    x = ALPHA_Q * cq + ALPHA_P * cp
    return jax.nn.log_softmax(x, axis=1)


# speedup vs baseline: 7.9338x; 1.0340x over previous
import functools

import jax
import jax.numpy as jnp
from jax import lax
from jax.experimental import pallas as pl
from jax.experimental.pallas import tpu as pltpu
from jax.experimental.pallas import tpu_sc as plsc

N = 10000
NP = 10240
NPT = NP // 16
E = 320000
F_IN = 128
HID = 256
NLAB = 64
EPT = E // 16
NB = 157
EPTP = NB * 128
HIDLAYERS = 3
ALPHA_Q = 0.7
ALPHA_P = 0.3

_MESH = plsc.VectorSubcoreMesh(core_axis_name="c", subcore_axis_name="s")


@functools.partial(
    pl.kernel,
    out_type=jax.ShapeDtypeStruct((2, NP, 16), jnp.float32),
    mesh=_MESH,
    compiler_params=pltpu.CompilerParams(use_tc_tiling_on_sc=False),
    scratch_types=[
        pltpu.VMEM((NB, 128), jnp.int32),
        pltpu.VMEM((5, 128), jnp.int32),
        pltpu.VMEM((128, 16), jnp.float32),
        pltpu.VMEM((NPT, 16), jnp.float32),
        pltpu.VMEM_SHARED((NP, 16), jnp.float32),
    ],
)
def _deg_kernel(dst_hbm, zi_hbm, deg_out, idx_v, zi_v, ones_v, stage_v, deg_sh):
    c = lax.axis_index("c")
    s = lax.axis_index("s")

    pltpu.sync_copy(dst_hbm.at[c, s], idx_v)
    pltpu.sync_copy(zi_hbm.at[s], zi_v)

    ones16 = jnp.ones((16,), jnp.float32)
    zeros16 = jnp.zeros((16,), jnp.float32)

    @pl.loop(0, 128)
    def _(i):
        ones_v[i, :] = ones16

    @pl.loop(0, NPT)
    def _(i):
        stage_v[i, :] = zeros16

    for k in range(5):
        pltpu.sync_copy(stage_v.at[pl.ds(0, 128)], deg_sh.at[zi_v.at[k]])
    plsc.subcore_barrier()

    @pl.loop(0, NB)
    def _(j):
        pltpu.sync_copy(ones_v, deg_sh.at[idx_v.at[j]], add=True)

    plsc.subcore_barrier()
    for k in range(5):
        pltpu.sync_copy(deg_sh.at[zi_v.at[k]], stage_v.at[pl.ds(k * 128, 128)])
    pltpu.sync_copy(stage_v, deg_out.at[c, pl.ds(s * NPT, NPT)])


def _make_conv(W, ROUNDS):
    CH = 2 * ROUNDS

    @functools.partial(
        pl.kernel,
        out_type=jax.ShapeDtypeStruct((CH, NP, W), jnp.float32),
        mesh=_MESH,
        compiler_params=pltpu.CompilerParams(use_tc_tiling_on_sc=False),
        scratch_types=[
            pltpu.VMEM((NB, 128), jnp.int32),
            pltpu.VMEM((NB, 128), jnp.int32),
            pltpu.VMEM((5, 128), jnp.int32),
            pltpu.VMEM((128, W), jnp.float32),
            pltpu.VMEM_SHARED((NP, W), jnp.float32),
        ],
    )
    def conv(h_hbm, src_hbm, dst_hbm, zi_hbm, out_hbm, src_v, dst_v, zi_v, buf_v, agg_sh):
        c = lax.axis_index("c")
        s = lax.axis_index("s")
        pltpu.sync_copy(dst_hbm.at[s], dst_v)
        pltpu.sync_copy(zi_hbm.at[s], zi_v)
        zeros16 = jnp.zeros((16,), jnp.float32)

        for r in range(ROUNDS):
            ch = c + 2 * r
            pltpu.sync_copy(src_hbm.at[ch, s], src_v)

            @pl.loop(0, 128)
            def _(i):
                for t in range(W // 16):
                    buf_v[i, pl.ds(t * 16, 16)] = zeros16

            for k in range(5):
                pltpu.sync_copy(buf_v, agg_sh.at[zi_v.at[k]])
            plsc.subcore_barrier()

            @pl.loop(0, NB)
            def _(j):
                pltpu.sync_copy(h_hbm.at[src_v.at[j]], buf_v)
                pltpu.sync_copy(buf_v, agg_sh.at[dst_v.at[j]], add=True)

            plsc.subcore_barrier()
            for k in range(5):
                pltpu.sync_copy(agg_sh.at[zi_v.at[k]], buf_v)
                pltpu.sync_copy(buf_v, out_hbm.at[ch, pl.ds(s * NPT + k * 128, 128)])
            if r + 1 < ROUNDS:
                plsc.subcore_barrier()

    return conv


_conv64 = _make_conv(64, 2)
_conv32 = _make_conv(32, 1)


def _pad_edge(e, fill):
    d = e.reshape(16, EPT)
    d = jnp.pad(d, ((0, 0), (0, EPTP - EPT)), constant_values=fill)
    return d.reshape(16, NB, 128)


BN = 512
NBLK = NP // BN


def _stats_body(x_ref, mu_ref, sd_ref):
    x = x_ref[...]
    mu = jnp.mean(x, axis=0, keepdims=True)
    d = x - mu
    var = jnp.sum(d * d, axis=0, keepdims=True) / (N - 1)
    mu_ref[...] = mu
    sd_ref[...] = jnp.sqrt(var)


def _tc_stats(x):
    return pl.pallas_call(
        _stats_body,
        out_shape=[jax.ShapeDtypeStruct((1, F_IN), jnp.float32),
                   jax.ShapeDtypeStruct((1, F_IN), jnp.float32)],
    )(x)


def _dinv_blk(deg_ref, g):
    return jax.lax.rsqrt(deg_ref[g][:, 0:1] + 1.0)


def _lin0_body(x_ref, mu_ref, sd_ref, deg_ref, wq_ref, wp_ref, hq_ref, hp_ref):
    xs = (x_ref[...] - mu_ref[...]) / sd_ref[...]
    dq = _dinv_blk(deg_ref, 0)
    dp = _dinv_blk(deg_ref, 1)
    hq = jnp.dot(xs, wq_ref[...], preferred_element_type=jnp.float32) * dq
    hp = jnp.dot(xs, wp_ref[...], preferred_element_type=jnp.float32) * dp
    for k in range(4):
        hq_ref[k] = hq[:, k * 64:(k + 1) * 64]
        hp_ref[k] = hp[:, k * 64:(k + 1) * 64]


def _tc_lin0(x_pad, mu, sd, deg, Wq, Wp):
    full = lambda shp: pl.BlockSpec(shp, lambda i: tuple(0 for _ in shp))
    return pl.pallas_call(
        _lin0_body,
        grid=(NBLK,),
        in_specs=[
            pl.BlockSpec((BN, F_IN), lambda i: (i, 0)),
            full((1, F_IN)), full((1, F_IN)),
            pl.BlockSpec((2, BN, 16), lambda i: (0, i, 0)),
            full((F_IN, HID)), full((F_IN, HID)),
        ],
        out_specs=[pl.BlockSpec((4, BN, 64), lambda i: (0, i, 0))] * 2,
        out_shape=[jax.ShapeDtypeStruct((4, NP, 64), jnp.float32)] * 2,
    )(x_pad, mu, sd, deg, Wq, Wp)


def _mid_body(aq_ref, ap_ref, hq_ref, hp_ref, deg_ref, bq_ref, bp_ref,
              wq_ref, wp_ref, oq_ref, op_ref, *, och, ow):
    dq = _dinv_blk(deg_ref, 0)
    dp = _dinv_blk(deg_ref, 1)
    aq = jnp.concatenate([aq_ref[k] for k in range(4)], axis=1)
    ap = jnp.concatenate([ap_ref[k] for k in range(4)], axis=1)
    hq = jnp.concatenate([hq_ref[k] for k in range(4)], axis=1)
    hp = jnp.concatenate([hp_ref[k] for k in range(4)], axis=1)
    cq = (aq + hq) * dq + bq_ref[...]
    cp = (ap + hp) * dp + bp_ref[...]
    x = ALPHA_Q * jax.nn.relu(cq) + ALPHA_P * jax.nn.relu(cp)
    oq = jnp.dot(x, wq_ref[...], preferred_element_type=jnp.float32) * dq
    op = jnp.dot(x, wp_ref[...], preferred_element_type=jnp.float32) * dp
    for k in range(och):
        oq_ref[k] = oq[:, k * ow:(k + 1) * ow]
        op_ref[k] = op[:, k * ow:(k + 1) * ow]


def _tc_mid(aggq, aggp, hq, hp, deg, bq, bp, Wq, Wp, och, ow):
    full = lambda shp: pl.BlockSpec(shp, lambda i: tuple(0 for _ in shp))
    wsh = Wq.shape
    return pl.pallas_call(
        functools.partial(_mid_body, och=och, ow=ow),
        grid=(NBLK,),
        in_specs=[
            pl.BlockSpec((4, BN, 64), lambda i: (0, i, 0)),
            pl.BlockSpec((4, BN, 64), lambda i: (0, i, 0)),
            pl.BlockSpec((4, BN, 64), lambda i: (0, i, 0)),
            pl.BlockSpec((4, BN, 64), lambda i: (0, i, 0)),
            pl.BlockSpec((2, BN, 16), lambda i: (0, i, 0)),
            full((1, HID)), full((1, HID)),
            full(wsh), full(wsh),
        ],
        out_specs=[pl.BlockSpec((och, BN, ow), lambda i: (0, i, 0))] * 2,
        out_shape=[jax.ShapeDtypeStruct((och, NP, ow), jnp.float32)] * 2,
    )(aggq, aggp, hq, hp, deg, bq, bp, Wq, Wp)


def _out_body(aq_ref, ap_ref, hq_ref, hp_ref, deg_ref, bq_ref, bp_ref, o_ref):
    dq = _dinv_blk(deg_ref, 0)
    dp = _dinv_blk(deg_ref, 1)
    aq = jnp.concatenate([aq_ref[k] for k in range(2)], axis=1)
    ap = jnp.concatenate([ap_ref[k] for k in range(2)], axis=1)
    hq = jnp.concatenate([hq_ref[k] for k in range(2)], axis=1)
    hp = jnp.concatenate([hp_ref[k] for k in range(2)], axis=1)
    cq = (aq + hq) * dq + bq_ref[...]
    cp = (ap + hp) * dp + bp_ref[...]
    x = ALPHA_Q * cq + ALPHA_P * cp
    m = jnp.max(x, axis=1, keepdims=True)
    lse = jnp.log(jnp.sum(jnp.exp(x - m), axis=1, keepdims=True))
    o_ref[...] = x - m - lse


def _tc_out(aggq, aggp, hq, hp, deg, bq, bp):
    full = lambda shp: pl.BlockSpec(shp, lambda i: tuple(0 for _ in shp))
    return pl.pallas_call(
        _out_body,
        grid=(NBLK,),
        in_specs=[
            pl.BlockSpec((2, BN, 32), lambda i: (0, i, 0)),
            pl.BlockSpec((2, BN, 32), lambda i: (0, i, 0)),
            pl.BlockSpec((2, BN, 32), lambda i: (0, i, 0)),
            pl.BlockSpec((2, BN, 32), lambda i: (0, i, 0)),
            pl.BlockSpec((2, BN, 16), lambda i: (0, i, 0)),
            full((1, NLAB)), full((1, NLAB)),
        ],
        out_specs=pl.BlockSpec((BN, NLAB), lambda i: (i, 0)),
        out_shape=jax.ShapeDtypeStruct((NP, NLAB), jnp.float32),
    )(aggq, aggp, hq, hp, deg, bq, bp)


def _sc_conv(h, dinv, src_off, dst_t, zi, conv_fn, W):
    CH = h.shape[1] // W
    hp = h * dinv[:, None]
    hp_pad = jnp.pad(hp, ((0, NP - N), (0, 0)))
    hchunk = jnp.concatenate([hp_pad[:, k * W:(k + 1) * W] for k in range(CH)], axis=0)
    agg = conv_fn(hchunk, src_off, dst_t, zi)
    agg_full = jnp.concatenate([agg[k, :N] for k in range(CH)], axis=1)
    return (agg_full + hp) * dinv[:, None]


def kernel(x, edge_index_q, edge_index_p, Wq0, bq0, Wp0, bp0, Wq1, bq1, Wp1, bp1, Wq2, bq2, Wp2, bp2):
    zi = jnp.arange(NP, dtype=jnp.int32).reshape(16, 5, 128)
    dst2 = jnp.stack([_pad_edge(edge_index_q[1], N), _pad_edge(edge_index_p[1], N)])
    deg = _deg_kernel(dst2, zi)

    sq = _pad_edge(edge_index_q[0], 0)
    sp = _pad_edge(edge_index_p[0], 0)
    src_q4 = jnp.stack([sq + k * NP for k in range(4)])
    src_p4 = jnp.stack([sp + k * NP for k in range(4)])
    dst_q = _pad_edge(edge_index_q[1], N)
    dst_p = _pad_edge(edge_index_p[1], N)

    bq0r, bp0r = bq0.reshape(1, HID), bp0.reshape(1, HID)
    bq1r, bp1r = bq1.reshape(1, HID), bp1.reshape(1, HID)
    bq2r, bp2r = bq2.reshape(1, NLAB), bp2.reshape(1, NLAB)

    mu, sd = _tc_stats(x)
    x_pad = jnp.pad(x, ((0, NP - N), (0, 0)))
    hq, hp = _tc_lin0(x_pad, mu, sd, deg, Wq0, Wp0)

    def agg_pair(hq, hp, conv_fn, src_q, src_p):
        aq = conv_fn(hq.reshape(-1, hq.shape[-1]), src_q, dst_q, zi)
        ap = conv_fn(hp.reshape(-1, hp.shape[-1]), src_p, dst_p, zi)
        return aq, ap

    aq, ap = agg_pair(hq, hp, _conv64, src_q4, src_p4)
    hq, hp = _tc_mid(aq, ap, hq, hp, deg, bq0r, bp0r, Wq1, Wp1, 4, 64)
    aq, ap = agg_pair(hq, hp, _conv64, src_q4, src_p4)
    hq, hp = _tc_mid(aq, ap, hq, hp, deg, bq1r, bp1r, Wq1, Wp1, 4, 64)
    aq, ap = agg_pair(hq, hp, _conv64, src_q4, src_p4)
    hq, hp = _tc_mid(aq, ap, hq, hp, deg, bq1r, bp1r, Wq2, Wp2, 2, 32)
    aq, ap = agg_pair(hq, hp, _conv32, src_q4[:2], src_p4[:2])
    out = _tc_out(aq, ap, hq, hp, deg, bq2r, bp2r)
    return out[:N]


# speedup vs baseline: 12.1545x; 1.5320x over previous
import functools

import jax
import jax.numpy as jnp
from jax import lax
from jax.experimental import pallas as pl
from jax.experimental.pallas import tpu as pltpu
from jax.experimental.pallas import tpu_sc as plsc

N = 10000
NP = 10240
NPT = NP // 16
E = 320000
F_IN = 128
HID = 256
NLAB = 64
EPT = E // 16
NB = 157
EPTP = NB * 128
HIDLAYERS = 3
ALPHA_Q = 0.7
ALPHA_P = 0.3

_MESH = plsc.VectorSubcoreMesh(core_axis_name="c", subcore_axis_name="s")


@functools.partial(
    pl.kernel,
    out_type=jax.ShapeDtypeStruct((2, NP, 16), jnp.float32),
    mesh=_MESH,
    compiler_params=pltpu.CompilerParams(use_tc_tiling_on_sc=False),
    scratch_types=[
        pltpu.VMEM((NB, 128), jnp.int32),
        pltpu.VMEM((5, 128), jnp.int32),
        pltpu.VMEM((128, 16), jnp.float32),
        pltpu.VMEM((NPT, 16), jnp.float32),
        pltpu.VMEM_SHARED((NP, 16), jnp.float32),
    ],
)
def _deg_kernel(dst_hbm, zi_hbm, deg_out, idx_v, zi_v, ones_v, stage_v, deg_sh):
    c = lax.axis_index("c")
    s = lax.axis_index("s")

    pltpu.sync_copy(dst_hbm.at[c, s], idx_v)
    pltpu.sync_copy(zi_hbm.at[s], zi_v)

    ones16 = jnp.ones((16,), jnp.float32)
    zeros16 = jnp.zeros((16,), jnp.float32)

    @pl.loop(0, 128)
    def _(i):
        ones_v[i, :] = ones16

    @pl.loop(0, NPT)
    def _(i):
        stage_v[i, :] = zeros16

    for k in range(5):
        pltpu.sync_copy(stage_v.at[pl.ds(0, 128)], deg_sh.at[zi_v.at[k]])
    plsc.subcore_barrier()

    @pl.loop(0, NB)
    def _(j):
        pltpu.sync_copy(ones_v, deg_sh.at[idx_v.at[j]], add=True)

    plsc.subcore_barrier()
    for k in range(5):
        pltpu.sync_copy(deg_sh.at[zi_v.at[k]], stage_v.at[pl.ds(k * 128, 128)])
    pltpu.sync_copy(stage_v, deg_out.at[c, pl.ds(s * NPT, NPT)])


def _make_conv(W, ROUNDS):
    CH = 2 * ROUNDS

    @functools.partial(
        pl.kernel,
        out_type=jax.ShapeDtypeStruct((CH, NP, W), jnp.float32),
        mesh=_MESH,
        compiler_params=pltpu.CompilerParams(use_tc_tiling_on_sc=False),
        scratch_types=[
            pltpu.VMEM((NB, 128), jnp.int32),
            pltpu.VMEM((NB, 128), jnp.int32),
            pltpu.VMEM((5, 128), jnp.int32),
            pltpu.VMEM((128, W), jnp.float32),
            pltpu.VMEM((128, W), jnp.float32),
            pltpu.SemaphoreType.DMA,
            pltpu.SemaphoreType.DMA,
            pltpu.VMEM_SHARED((NP, W), jnp.float32),
        ],
    )
    def conv(h_hbm, src_hbm, dst_hbm, zi_hbm, out_hbm, src_v, dst_v, zi_v, buf_v, buf2_v, sem0, sem1, agg_sh):
        c = lax.axis_index("c")
        s = lax.axis_index("s")
        pltpu.sync_copy(dst_hbm.at[s], dst_v)
        pltpu.sync_copy(zi_hbm.at[s], zi_v)
        zeros16 = jnp.zeros((16,), jnp.float32)

        for r in range(ROUNDS):
            ch = c + 2 * r
            pltpu.sync_copy(src_hbm.at[ch, s], src_v)

            @pl.loop(0, 128)
            def _(i):
                for t in range(W // 16):
                    buf_v[i, pl.ds(t * 16, 16)] = zeros16

            for k in range(5):
                pltpu.sync_copy(buf_v, agg_sh.at[zi_v.at[k]])
            plsc.subcore_barrier()

            pltpu.async_copy(h_hbm.at[src_v.at[0]], buf_v, sem0)

            @pl.loop(0, (NB - 1) // 2)
            def _(p):
                j0 = 2 * p
                pltpu.async_copy(h_hbm.at[src_v.at[j0 + 1]], buf2_v, sem1)
                pltpu.make_async_copy(h_hbm.at[src_v.at[j0]], buf_v, sem0).wait()
                pltpu.sync_copy(buf_v, agg_sh.at[dst_v.at[j0]], add=True)
                pltpu.async_copy(h_hbm.at[src_v.at[j0 + 2]], buf_v, sem0)
                pltpu.make_async_copy(h_hbm.at[src_v.at[j0 + 1]], buf2_v, sem1).wait()
                pltpu.sync_copy(buf2_v, agg_sh.at[dst_v.at[j0 + 1]], add=True)

            pltpu.make_async_copy(h_hbm.at[src_v.at[NB - 1]], buf_v, sem0).wait()
            pltpu.sync_copy(buf_v, agg_sh.at[dst_v.at[NB - 1]], add=True)

            plsc.subcore_barrier()
            for k in range(5):
                pltpu.sync_copy(agg_sh.at[zi_v.at[k]], buf_v)
                pltpu.sync_copy(buf_v, out_hbm.at[ch, pl.ds(s * NPT + k * 128, 128)])
            if r + 1 < ROUNDS:
                plsc.subcore_barrier()

    return conv


_conv64 = _make_conv(64, 2)
_conv32 = _make_conv(32, 1)


def _pad_edge(e, fill):
    d = e.reshape(16, EPT)
    d = jnp.pad(d, ((0, 0), (0, EPTP - EPT)), constant_values=fill)
    return d.reshape(16, NB, 128)


BN = 512
NBLK = NP // BN


def _stats_body(x_ref, mu_ref, sd_ref):
    x = x_ref[...]
    mu = jnp.mean(x, axis=0, keepdims=True)
    d = x - mu
    var = jnp.sum(d * d, axis=0, keepdims=True) / (N - 1)
    mu_ref[...] = mu
    sd_ref[...] = jnp.sqrt(var)


def _tc_stats(x):
    return pl.pallas_call(
        _stats_body,
        out_shape=[jax.ShapeDtypeStruct((1, F_IN), jnp.float32),
                   jax.ShapeDtypeStruct((1, F_IN), jnp.float32)],
    )(x)


def _dinv_blk(deg_ref, g):
    return jax.lax.rsqrt(deg_ref[g][:, 0:1] + 1.0)


def _lin0_body(x_ref, mu_ref, sd_ref, deg_ref, wq_ref, wp_ref, hq_ref, hp_ref):
    xs = (x_ref[...] - mu_ref[...]) / sd_ref[...]
    dq = _dinv_blk(deg_ref, 0)
    dp = _dinv_blk(deg_ref, 1)
    hq = jnp.dot(xs, wq_ref[...], preferred_element_type=jnp.float32) * dq
    hp = jnp.dot(xs, wp_ref[...], preferred_element_type=jnp.float32) * dp
    for k in range(4):
        hq_ref[k] = hq[:, k * 64:(k + 1) * 64]
        hp_ref[k] = hp[:, k * 64:(k + 1) * 64]


def _tc_lin0(x_pad, mu, sd, deg, Wq, Wp):
    full = lambda shp: pl.BlockSpec(shp, lambda i: tuple(0 for _ in shp))
    return pl.pallas_call(
        _lin0_body,
        grid=(NBLK,),
        in_specs=[
            pl.BlockSpec((BN, F_IN), lambda i: (i, 0)),
            full((1, F_IN)), full((1, F_IN)),
            pl.BlockSpec((2, BN, 16), lambda i: (0, i, 0)),
            full((F_IN, HID)), full((F_IN, HID)),
        ],
        out_specs=[pl.BlockSpec((4, BN, 64), lambda i: (0, i, 0))] * 2,
        out_shape=[jax.ShapeDtypeStruct((4, NP, 64), jnp.float32)] * 2,
    )(x_pad, mu, sd, deg, Wq, Wp)


def _mid_body(aq_ref, ap_ref, hq_ref, hp_ref, deg_ref, bq_ref, bp_ref,
              wq_ref, wp_ref, oq_ref, op_ref, *, och, ow):
    dq = _dinv_blk(deg_ref, 0)
    dp = _dinv_blk(deg_ref, 1)
    aq = jnp.concatenate([aq_ref[k] for k in range(4)], axis=1)
    ap = jnp.concatenate([ap_ref[k] for k in range(4)], axis=1)
    hq = jnp.concatenate([hq_ref[k] for k in range(4)], axis=1)
    hp = jnp.concatenate([hp_ref[k] for k in range(4)], axis=1)
    cq = (aq + hq) * dq + bq_ref[...]
    cp = (ap + hp) * dp + bp_ref[...]
    x = ALPHA_Q * jax.nn.relu(cq) + ALPHA_P * jax.nn.relu(cp)
    oq = jnp.dot(x, wq_ref[...], preferred_element_type=jnp.float32) * dq
    op = jnp.dot(x, wp_ref[...], preferred_element_type=jnp.float32) * dp
    for k in range(och):
        oq_ref[k] = oq[:, k * ow:(k + 1) * ow]
        op_ref[k] = op[:, k * ow:(k + 1) * ow]


def _tc_mid(aggq, aggp, hq, hp, deg, bq, bp, Wq, Wp, och, ow):
    full = lambda shp: pl.BlockSpec(shp, lambda i: tuple(0 for _ in shp))
    wsh = Wq.shape
    return pl.pallas_call(
        functools.partial(_mid_body, och=och, ow=ow),
        grid=(NBLK,),
        in_specs=[
            pl.BlockSpec((4, BN, 64), lambda i: (0, i, 0)),
            pl.BlockSpec((4, BN, 64), lambda i: (0, i, 0)),
            pl.BlockSpec((4, BN, 64), lambda i: (0, i, 0)),
            pl.BlockSpec((4, BN, 64), lambda i: (0, i, 0)),
            pl.BlockSpec((2, BN, 16), lambda i: (0, i, 0)),
            full((1, HID)), full((1, HID)),
            full(wsh), full(wsh),
        ],
        out_specs=[pl.BlockSpec((och, BN, ow), lambda i: (0, i, 0))] * 2,
        out_shape=[jax.ShapeDtypeStruct((och, NP, ow), jnp.float32)] * 2,
    )(aggq, aggp, hq, hp, deg, bq, bp, Wq, Wp)


def _out_body(aq_ref, ap_ref, hq_ref, hp_ref, deg_ref, bq_ref, bp_ref, o_ref):
    dq = _dinv_blk(deg_ref, 0)
    dp = _dinv_blk(deg_ref, 1)
    aq = jnp.concatenate([aq_ref[k] for k in range(2)], axis=1)
    ap = jnp.concatenate([ap_ref[k] for k in range(2)], axis=1)
    hq = jnp.concatenate([hq_ref[k] for k in range(2)], axis=1)
    hp = jnp.concatenate([hp_ref[k] for k in range(2)], axis=1)
    cq = (aq + hq) * dq + bq_ref[...]
    cp = (ap + hp) * dp + bp_ref[...]
    x = ALPHA_Q * cq + ALPHA_P * cp
    m = jnp.max(x, axis=1, keepdims=True)
    lse = jnp.log(jnp.sum(jnp.exp(x - m), axis=1, keepdims=True))
    o_ref[...] = x - m - lse


def _tc_out(aggq, aggp, hq, hp, deg, bq, bp):
    full = lambda shp: pl.BlockSpec(shp, lambda i: tuple(0 for _ in shp))
    return pl.pallas_call(
        _out_body,
        grid=(NBLK,),
        in_specs=[
            pl.BlockSpec((2, BN, 32), lambda i: (0, i, 0)),
            pl.BlockSpec((2, BN, 32), lambda i: (0, i, 0)),
            pl.BlockSpec((2, BN, 32), lambda i: (0, i, 0)),
            pl.BlockSpec((2, BN, 32), lambda i: (0, i, 0)),
            pl.BlockSpec((2, BN, 16), lambda i: (0, i, 0)),
            full((1, NLAB)), full((1, NLAB)),
        ],
        out_specs=pl.BlockSpec((BN, NLAB), lambda i: (i, 0)),
        out_shape=jax.ShapeDtypeStruct((NP, NLAB), jnp.float32),
    )(aggq, aggp, hq, hp, deg, bq, bp)


def _sc_conv(h, dinv, src_off, dst_t, zi, conv_fn, W):
    CH = h.shape[1] // W
    hp = h * dinv[:, None]
    hp_pad = jnp.pad(hp, ((0, NP - N), (0, 0)))
    hchunk = jnp.concatenate([hp_pad[:, k * W:(k + 1) * W] for k in range(CH)], axis=0)
    agg = conv_fn(hchunk, src_off, dst_t, zi)
    agg_full = jnp.concatenate([agg[k, :N] for k in range(CH)], axis=1)
    return (agg_full + hp) * dinv[:, None]


def kernel(x, edge_index_q, edge_index_p, Wq0, bq0, Wp0, bp0, Wq1, bq1, Wp1, bp1, Wq2, bq2, Wp2, bp2):
    zi = jnp.arange(NP, dtype=jnp.int32).reshape(16, 5, 128)
    dst2 = jnp.stack([_pad_edge(edge_index_q[1], N), _pad_edge(edge_index_p[1], N)])
    deg = _deg_kernel(dst2, zi)

    sq = _pad_edge(edge_index_q[0], 0)
    sp = _pad_edge(edge_index_p[0], 0)
    src_q4 = jnp.stack([sq + k * NP for k in range(4)])
    src_p4 = jnp.stack([sp + k * NP for k in range(4)])
    dst_q = _pad_edge(edge_index_q[1], N)
    dst_p = _pad_edge(edge_index_p[1], N)

    bq0r, bp0r = bq0.reshape(1, HID), bp0.reshape(1, HID)
    bq1r, bp1r = bq1.reshape(1, HID), bp1.reshape(1, HID)
    bq2r, bp2r = bq2.reshape(1, NLAB), bp2.reshape(1, NLAB)

    mu, sd = _tc_stats(x)
    x_pad = jnp.pad(x, ((0, NP - N), (0, 0)))
    hq, hp = _tc_lin0(x_pad, mu, sd, deg, Wq0, Wp0)

    def agg_pair(hq, hp, conv_fn, src_q, src_p):
        aq = conv_fn(hq.reshape(-1, hq.shape[-1]), src_q, dst_q, zi)
        ap = conv_fn(hp.reshape(-1, hp.shape[-1]), src_p, dst_p, zi)
        return aq, ap

    aq, ap = agg_pair(hq, hp, _conv64, src_q4, src_p4)
    hq, hp = _tc_mid(aq, ap, hq, hp, deg, bq0r, bp0r, Wq1, Wp1, 4, 64)
    aq, ap = agg_pair(hq, hp, _conv64, src_q4, src_p4)
    hq, hp = _tc_mid(aq, ap, hq, hp, deg, bq1r, bp1r, Wq1, Wp1, 4, 64)
    aq, ap = agg_pair(hq, hp, _conv64, src_q4, src_p4)
    hq, hp = _tc_mid(aq, ap, hq, hp, deg, bq1r, bp1r, Wq2, Wp2, 2, 32)
    aq, ap = agg_pair(hq, hp, _conv32, src_q4[:2], src_p4[:2])
    out = _tc_out(aq, ap, hq, hp, deg, bq2r, bp2r)
    return out[:N]
